# Initial kernel scaffold; baseline (speedup 1.0000x reference)
import jax
import jax.numpy as jnp
from jax.experimental import pallas as pl


def kernel(x_seq, u_seq, logP0, ws):
    raise NotImplementedError("write your pallas kernel here")


# speedup vs baseline: 2.5032x; 2.5032x over previous
import functools

import jax
import jax.numpy as jnp
from jax import lax
from jax.experimental import pallas as pl
from jax.experimental.pallas import tpu as pltpu


def _prep_body(lpt_ref, p0t_ref, l_ref):
    x = lpt_ref[...]
    p0t_ref[...] = jnp.exp(x).astype(jnp.bfloat16)
    l_ref[...] = x.astype(jnp.bfloat16)


def _main_body(n_valid, u_ref, xc_ref, xn_ref, wst_ref, p0t_ref, l_ref, out_ref):
    i = pl.program_id(0)
    corr = jnp.dot(u_ref[...], wst_ref[...], preferred_element_type=jnp.float32)
    lane = lax.broadcasted_iota(jnp.int32, corr.shape, 1)
    corr = jnp.where(lane < n_valid, corr, -1e30)
    m = jnp.max(corr, axis=1, keepdims=True)
    e = jnp.exp(corr - m).astype(jnp.bfloat16)
    xc = xc_ref[...]
    xn = xn_ref[...]
    ohn = lane == xn
    mz = jnp.dot(e, p0t_ref[...], preferred_element_type=jnp.float32)
    msel = jnp.dot(ohn.astype(jnp.bfloat16), l_ref[...],
                   preferred_element_type=jnp.float32)
    ohc = lane == xc
    z = jnp.sum(jnp.where(ohc, mz, 0.0), axis=1)
    selp = jnp.sum(jnp.where(ohc, msel, 0.0), axis=1)
    selc = jnp.sum(jnp.where(ohn, corr, 0.0), axis=1)
    block = jnp.sum(selc + selp - m[:, 0] - jnp.log(z))

    @pl.when(i == 0)
    def _():
        out_ref[...] = jnp.zeros((1, 1), jnp.float32)

    out_ref[...] += block.reshape(1, 1)


def kernel(x_seq, u_seq, logP0, ws):
    n = logP0.shape[0]
    u_dim = u_seq.shape[1]
    t1 = x_seq.shape[0] - 1
    npad = ((n + 127) // 128) * 128
    bt = 512
    while t1 % bt != 0:
        bt //= 2
    grid = t1 // bt

    xc = x_seq[:-1].astype(jnp.int32).reshape(t1, 1)
    xn = x_seq[1:].astype(jnp.int32).reshape(t1, 1)
    u = u_seq[:-1].astype(jnp.bfloat16)
    wst = jnp.pad(ws, ((0, npad - n), (0, 0))).T.astype(jnp.bfloat16)
    lpt = jnp.pad(logP0, ((0, npad - n), (0, npad - n)),
                  constant_values=-1e30).T

    p0t, ltab = pl.pallas_call(
        _prep_body,
        out_shape=(
            jax.ShapeDtypeStruct((npad, npad), jnp.bfloat16),
            jax.ShapeDtypeStruct((npad, npad), jnp.bfloat16),
        ),
    )(lpt)

    out = pl.pallas_call(
        functools.partial(_main_body, n),
        grid=(grid,),
        in_specs=[
            pl.BlockSpec((bt, u_dim), lambda i: (i, 0)),
            pl.BlockSpec((bt, 1), lambda i: (i, 0)),
            pl.BlockSpec((bt, 1), lambda i: (i, 0)),
            pl.BlockSpec((u_dim, npad), lambda i: (0, 0)),
            pl.BlockSpec((npad, npad), lambda i: (0, 0)),
            pl.BlockSpec((npad, npad), lambda i: (0, 0)),
        ],
        out_specs=pl.BlockSpec((1, 1), lambda i: (0, 0)),
        out_shape=jax.ShapeDtypeStruct((1, 1), jnp.float32),
        compiler_params=pltpu.CompilerParams(
            dimension_semantics=("arbitrary",),
        ),
    )(u, xc, xn, wst, p0t, ltab)

    return -out[0, 0]


# speedup vs baseline: 3.4174x; 1.3652x over previous
import dataclasses
import functools

import jax
import jax.numpy as jnp
from jax import lax
from jax.experimental import pallas as pl
from jax.experimental.pallas import tpu as pltpu
from jax.experimental.pallas import tpu_sc as plsc

_W = 128


def _prep_body(lpt_ref, p0t_ref):
    p0t_ref[...] = jnp.exp(lpt_ref[...]).astype(jnp.bfloat16)


def _sc_selected_sum(tab128, ridx, lane):
    t1 = ridx.shape[0]
    mesh = plsc.VectorSubcoreMesh(core_axis_name="c", subcore_axis_name="s")
    cp = pltpu.CompilerParams()
    if "needs_layout_passes" in pltpu.CompilerParams.__dataclass_fields__:
        cp = dataclasses.replace(cp, needs_layout_passes=False)

    @functools.partial(
        pl.kernel,
        out_type=jax.ShapeDtypeStruct((2, 16, 16), jnp.float32),
        mesh=mesh,
        compiler_params=cp,
        scratch_types=[
            pltpu.VMEM((_W, 128), jnp.float32),
            pltpu.VMEM((16,), jnp.float32),
        ],
    )
    def sel_kernel(tab_hbm, ridx_hbm, lane_hbm, out_hbm, g_ref, acc_ref):
        c = lax.axis_index("c")
        s = lax.axis_index("s")
        acc_ref[...] = jnp.zeros((16,), jnp.float32)

        def body(ridx_vmem, lane_vmem):
            pltpu.sync_copy(tab_hbm.at[ridx_vmem.at[0]], g_ref)

            @pl.loop(0, _W, step=16)
            def _(i):
                rows = lax.iota(jnp.int32, 16) + i
                lanes = lane_vmem[0, pl.ds(i, 16)]
                acc_ref[...] += plsc.load_gather(g_ref, [rows, lanes])

        pltpu.emit_pipeline(
            body,
            grid=(t1 // _W,),
            in_specs=[
                pl.BlockSpec((1, _W), index_map=lambda i: (0, i)),
                pl.BlockSpec((1, _W), index_map=lambda i: (0, i)),
            ],
            out_specs=[],
            core_axis_name=("c", "s"),
            dimension_semantics=(pltpu.PARALLEL,),
        )(ridx_hbm, lane_hbm)

        pltpu.sync_copy(acc_ref, out_hbm.at[c, s])

    return sel_kernel(tab128, ridx.reshape(1, t1), lane.reshape(1, t1))


def _main_body(n_valid, u_ref, xc_ref, xn_ref, wst_ref, p0t_ref, out_ref):
    i = pl.program_id(0)
    corr = jnp.dot(u_ref[...], wst_ref[...], preferred_element_type=jnp.float32)
    lane = lax.broadcasted_iota(jnp.int32, corr.shape, 1)
    corr = jnp.where(lane < n_valid, corr, -1e30)
    m = jnp.max(corr, axis=1, keepdims=True)
    e = jnp.exp(corr - m).astype(jnp.bfloat16)
    xc = xc_ref[...]
    xn = xn_ref[...]
    mz = jnp.dot(e, p0t_ref[...], preferred_element_type=jnp.float32)
    ohc = lane == xc
    z = jnp.sum(jnp.where(ohc, mz, 0.0), axis=1)
    selc = jnp.sum(jnp.where(lane == xn, corr, 0.0), axis=1)
    block = jnp.sum(selc - m[:, 0] - jnp.log(z))

    @pl.when(i == 0)
    def _():
        out_ref[...] = jnp.zeros((1, 1), jnp.float32)

    out_ref[...] += block.reshape(1, 1)


def kernel(x_seq, u_seq, logP0, ws):
    n = logP0.shape[0]
    u_dim = u_seq.shape[1]
    t1 = x_seq.shape[0] - 1
    npad = ((n + 127) // 128) * 128
    bt = 512
    while t1 % bt != 0:
        bt //= 2
    grid = t1 // bt

    xc = x_seq[:-1].astype(jnp.int32)
    xn = x_seq[1:].astype(jnp.int32)
    u = u_seq[:-1].astype(jnp.bfloat16)
    wst = jnp.pad(ws, ((0, npad - n), (0, 0))).T.astype(jnp.bfloat16)
    lp_pad = jnp.pad(logP0, ((0, npad - n), (0, npad - n)),
                     constant_values=-1e30)

    tab128 = lp_pad.reshape(npad * npad // 128, 128)
    flat = xc * npad + xn
    sc_partials = _sc_selected_sum(tab128, flat // 128, flat % 128)

    p0t = pl.pallas_call(
        _prep_body,
        out_shape=jax.ShapeDtypeStruct((npad, npad), jnp.bfloat16),
    )(lp_pad.T)

    out = pl.pallas_call(
        functools.partial(_main_body, n),
        grid=(grid,),
        in_specs=[
            pl.BlockSpec((bt, u_dim), lambda i: (i, 0)),
            pl.BlockSpec((bt, 1), lambda i: (i, 0)),
            pl.BlockSpec((bt, 1), lambda i: (i, 0)),
            pl.BlockSpec((u_dim, npad), lambda i: (0, 0)),
            pl.BlockSpec((npad, npad), lambda i: (0, 0)),
        ],
        out_specs=pl.BlockSpec((1, 1), lambda i: (0, 0)),
        out_shape=jax.ShapeDtypeStruct((1, 1), jnp.float32),
        compiler_params=pltpu.CompilerParams(
            dimension_semantics=("arbitrary",),
        ),
    )(u, xc.reshape(t1, 1), xn.reshape(t1, 1), wst, p0t)

    return -(out[0, 0] + jnp.sum(sc_partials))


# speedup vs baseline: 4.0616x; 1.1885x over previous
import dataclasses
import functools

import jax
import jax.numpy as jnp
from jax import lax
from jax.experimental import pallas as pl
from jax.experimental.pallas import tpu as pltpu
from jax.experimental.pallas import tpu_sc as plsc

_W = 128


def _prep_body(lpt_ref, p0t_ref):
    p0t_ref[...] = jnp.exp(lpt_ref[...]).astype(jnp.bfloat16)


def _sc_selected_sum(tab128, ridx, lane):
    t1 = ridx.shape[0]
    mesh = plsc.VectorSubcoreMesh(core_axis_name="c", subcore_axis_name="s")
    cp = pltpu.CompilerParams()
    if "needs_layout_passes" in pltpu.CompilerParams.__dataclass_fields__:
        cp = dataclasses.replace(cp, needs_layout_passes=False)

    @functools.partial(
        pl.kernel,
        out_type=jax.ShapeDtypeStruct((2, 16, 16), jnp.float32),
        mesh=mesh,
        compiler_params=cp,
        scratch_types=[
            pltpu.VMEM((_W, 128), jnp.float32),
            pltpu.VMEM((16,), jnp.float32),
        ],
    )
    def sel_kernel(tab_hbm, ridx_hbm, lane_hbm, out_hbm, g_ref, acc_ref):
        c = lax.axis_index("c")
        s = lax.axis_index("s")
        acc_ref[...] = jnp.zeros((16,), jnp.float32)

        def body(ridx_vmem, lane_vmem):
            pltpu.sync_copy(tab_hbm.at[ridx_vmem.at[0]], g_ref)

            @pl.loop(0, _W, step=16)
            def _(i):
                rows = lax.iota(jnp.int32, 16) + i
                lanes = lane_vmem[0, pl.ds(i, 16)]
                acc_ref[...] += plsc.load_gather(g_ref, [rows, lanes])

        pltpu.emit_pipeline(
            body,
            grid=(t1 // _W,),
            in_specs=[
                pl.BlockSpec((1, _W), index_map=lambda i: (0, i)),
                pl.BlockSpec((1, _W), index_map=lambda i: (0, i)),
            ],
            out_specs=[],
            core_axis_name=("c", "s"),
            dimension_semantics=(pltpu.PARALLEL,),
        )(ridx_hbm, lane_hbm)

        pltpu.sync_copy(acc_ref, out_hbm.at[c, s])

    return sel_kernel(tab128, ridx.reshape(1, t1), lane.reshape(1, t1))


def _main_body(u_ref, xc_ref, xn_ref, wst_ref, p0t_ref, out_ref):
    i = pl.program_id(0)
    corr = jnp.dot(u_ref[...].astype(jnp.bfloat16), wst_ref[...],
                   preferred_element_type=jnp.float32)
    lane = lax.broadcasted_iota(jnp.int32, corr.shape, 1)
    m = jnp.max(corr, axis=1, keepdims=True)
    e = jnp.exp(corr - m).astype(jnp.bfloat16)
    xc = xc_ref[...]
    xn = xn_ref[...]
    mz = jnp.dot(e, p0t_ref[...], preferred_element_type=jnp.float32)
    ohc = lane == xc
    z = jnp.sum(jnp.where(ohc, mz, 0.0), axis=1)
    selc = jnp.sum(jnp.where(lane == xn, corr, 0.0), axis=1)
    block = jnp.sum(selc - m[:, 0] - jnp.log(z))

    @pl.when(i == 0)
    def _():
        out_ref[...] = jnp.zeros((1, 1), jnp.float32)

    out_ref[...] += block.reshape(1, 1)


def kernel(x_seq, u_seq, logP0, ws):
    n = logP0.shape[0]
    u_dim = u_seq.shape[1]
    t1 = x_seq.shape[0] - 1
    npad = ((n + 127) // 128) * 128
    bt = 1024
    while t1 % bt != 0:
        bt //= 2
    grid = t1 // bt

    xc = x_seq[:-1].astype(jnp.int32)
    xn = x_seq[1:].astype(jnp.int32)
    wst = jnp.pad(ws, ((0, npad - n), (0, 0))).T.astype(jnp.bfloat16)
    lp_pad = jnp.pad(logP0, ((0, npad - n), (0, npad - n)),
                     constant_values=-1e30)

    tab128 = lp_pad.reshape(npad * npad // 128, 128)
    flat = xc * npad + xn
    sc_partials = _sc_selected_sum(tab128, flat // 128, flat % 128)

    p0t = pl.pallas_call(
        _prep_body,
        out_shape=jax.ShapeDtypeStruct((npad, npad), jnp.bfloat16),
    )(lp_pad.T)

    out = pl.pallas_call(
        _main_body,
        grid=(grid,),
        in_specs=[
            pl.BlockSpec((bt, u_dim), lambda i: (i, 0)),
            pl.BlockSpec((bt, 1), lambda i: (i, 0)),
            pl.BlockSpec((bt, 1), lambda i: (i, 0)),
            pl.BlockSpec((u_dim, npad), lambda i: (0, 0)),
            pl.BlockSpec((npad, npad), lambda i: (0, 0)),
        ],
        out_specs=pl.BlockSpec((1, 1), lambda i: (0, 0)),
        out_shape=jax.ShapeDtypeStruct((1, 1), jnp.float32),
        compiler_params=pltpu.CompilerParams(
            dimension_semantics=("arbitrary",),
        ),
    )(u_seq, xc.reshape(t1, 1), xn.reshape(t1, 1), wst, p0t)

    return -(out[0, 0] + jnp.sum(sc_partials))
